# baseline (speedup 1.0000x reference)
import functools

import jax
import jax.numpy as jnp
from jax import lax
from jax.experimental import pallas as pl
from jax.experimental.pallas import tpu as pltpu
from jax.experimental.pallas import tpu_sc as plsc

_L = 16


def _make_lookup(NB, S, V, D, NC, NS):
    NW = NC * NS
    OBW = NB // NW
    JT = OBW // 128
    TI = D // 8
    TJ = NB // 128
    assert NB % (NW * 128) == 0 and D % 8 == 0

    mesh = plsc.VectorSubcoreMesh(core_axis_name="c", subcore_axis_name="s")

    @functools.partial(
        pl.kernel,
        out_type=jax.ShapeDtypeStruct((S, TI, TJ, 8, 128), jnp.float32),
        mesh=mesh,
        scratch_types=[
            pltpu.VMEM((2, OBW), jnp.int32),
            pltpu.VMEM((2, JT, 128), jnp.int32),
            pltpu.VMEM((2, OBW + _L), jnp.int32),
            pltpu.VMEM((2, OBW, D), jnp.float32),
            pltpu.VMEM((2, TI, JT, 8, 128), jnp.float32),
            pltpu.SMEM((2,), jnp.int32),
            pltpu.SemaphoreType.DMA((2,)),
            pltpu.SemaphoreType.DMA((2,)),
            pltpu.SemaphoreType.DMA((2,)),
        ],
        compiler_params=pltpu.CompilerParams(
            use_tc_tiling_on_sc=False, needs_layout_passes=False),
    )
    def lookup(idx_hbm, table_hbm, out_hbm,
               idx_v, safe_v, oov_v, rows_v, trans_v, cnt_sm,
               semi, semg, semo):
        wid = lax.axis_index("s") * NC + lax.axis_index("c")
        lanes = lax.iota(jnp.int32, _L)
        zrow = jnp.zeros((_L,), jnp.float32)

        def stage_a(g):
            par = g % 2
            pltpu.async_copy(idx_hbm.at[g, pl.ds(wid * OBW, OBW)],
                             idx_v.at[par], semi.at[par])

        def stage_b(g):
            par = g % 2
            pltpu.make_async_copy(idx_hbm.at[g, pl.ds(wid * OBW, OBW)],
                                  idx_v.at[par], semi.at[par]).wait()

            @plsc.parallel_loop(0, OBW // _L, carry=jnp.int32(0))
            def fix(t, cnt):
                v = idx_v[par, pl.ds(t * _L, _L)]
                valid = v < V
                safe_v[par, t // 8, pl.ds((t % 8) * _L, _L)] = (
                    jnp.where(valid, v, 0))
                bad = ~valid
                pos = t * _L + lanes
                bc = jnp.cumsum(jnp.where(bad, 1, 0))
                plsc.store_scatter(oov_v.at[par], [cnt + bc - 1], pos,
                                   mask=bad)
                return cnt + bc[_L - 1]

            cnt_sm[par] = fix
            for j in range(JT):
                pltpu.async_copy(
                    table_hbm.at[safe_v.at[par, j]],
                    rows_v.at[par, pl.ds(j * 128, 128)],
                    semg.at[par])

        def stage_c(g, first):
            par = g % 2
            cnt = cnt_sm[par]
            for j in range(JT):
                pltpu.make_async_copy(
                    table_hbm.at[safe_v.at[par, j]],
                    rows_v.at[par, pl.ds(j * 128, 128)],
                    semg.at[par]).wait()

            def zero_rows(t, _z):
                lm = (t * _L + lanes) < cnt
                p = oov_v[par, pl.ds(t * _L, _L)]
                p = jnp.where(lm, p, 0)

                def zero_col(c, _c):
                    cc = jnp.full((_L,), c, jnp.int32)
                    plsc.store_scatter(rows_v.at[par], [p, cc], zrow,
                                       mask=lm)
                    return 0

                return lax.fori_loop(0, D, zero_col, 0)

            lax.fori_loop(0, (cnt + _L - 1) // _L, zero_rows, 0)

            @pl.when(~first)
            def _drain():
                for ti in range(TI):
                    pltpu.make_async_copy(
                        trans_v.at[par, ti],
                        out_hbm.at[g, ti, pl.ds(wid * JT, JT)],
                        semo.at[par]).wait()

            @plsc.parallel_loop(0, OBW // _L, unroll=2)
            def _tpose(t):
                bvec = t * _L + lanes
                j = t // 8
                kk = (t % 8) * _L
                for d in range(D):
                    dd = jnp.full((_L,), d, jnp.int32)
                    vals = plsc.load_gather(rows_v.at[par], [bvec, dd])
                    trans_v[par, d // 8, j, d % 8, pl.ds(kk, _L)] = vals

            for ti in range(TI):
                pltpu.async_copy(trans_v.at[par, ti],
                                 out_hbm.at[g, ti, pl.ds(wid * JT, JT)],
                                 semo.at[par])

        stage_a(0)
        stage_a(1)
        stage_b(0)

        def body(g, _):
            @pl.when(g + 2 < S)
            def _a():
                stage_a(g + 2)

            @pl.when(g + 1 < S)
            def _b():
                stage_b(g + 1)

            stage_c(g, g < 2)
            return 0

        lax.fori_loop(0, S, body, 0)

        for g in (S - 2, S - 1):
            par = g % 2
            for ti in range(TI):
                pltpu.make_async_copy(
                    trans_v.at[par, ti],
                    out_hbm.at[g, ti, pl.ds(wid * JT, JT)],
                    semo.at[par]).wait()

    return lookup


def kernel(indices, table):
    V, D = table.shape
    NB, S = indices.shape
    info = plsc.get_sparse_core_info()
    idx_sm = indices.T.astype(jnp.int32)
    lookup = _make_lookup(NB, S, V, D, info.num_cores, info.num_subcores)
    tiles = lookup(idx_sm, table)
    out = tiles.transpose((2, 4, 0, 1, 3)).reshape(NB, S, D)
    return out


# speedup vs baseline: 1.0045x; 1.0045x over previous
import functools

import jax
import jax.numpy as jnp
from jax import lax
from jax.experimental import pallas as pl
from jax.experimental.pallas import tpu as pltpu
from jax.experimental.pallas import tpu_sc as plsc

_L = 16


def _make_lookup(NB, S, V, D, NC, NS):
    NW = NC * NS
    OBW = NB // NW
    JT = OBW // 128
    TI = D // 8
    TJ = NB // 128
    assert NB % (NW * 128) == 0 and D % 8 == 0

    mesh = plsc.VectorSubcoreMesh(core_axis_name="c", subcore_axis_name="s")

    @functools.partial(
        pl.kernel,
        out_type=jax.ShapeDtypeStruct((S, TI, TJ, 8, 128), jnp.float32),
        mesh=mesh,
        scratch_types=[
            pltpu.VMEM((2, OBW), jnp.int32),
            pltpu.VMEM((2, JT, 128), jnp.int32),
            pltpu.VMEM((2, OBW + _L), jnp.int32),
            pltpu.VMEM((2, OBW, D), jnp.float32),
            pltpu.VMEM((2, TI, JT, 8, 128), jnp.float32),
            pltpu.SMEM((2,), jnp.int32),
            pltpu.SemaphoreType.DMA((2,)),
            pltpu.SemaphoreType.DMA((2,)),
            pltpu.SemaphoreType.DMA((2,)),
        ],
        compiler_params=pltpu.CompilerParams(
            use_tc_tiling_on_sc=False, needs_layout_passes=False),
    )
    def lookup(idx_hbm, table_hbm, out_hbm,
               idx_v, safe_v, oov_v, rows_v, trans_v, cnt_sm,
               semi, semg, semo):
        wid = lax.axis_index("s") * NC + lax.axis_index("c")
        lanes = lax.iota(jnp.int32, _L)
        zrow = jnp.zeros((_L,), jnp.float32)

        def stage_a(g):
            par = g % 2
            pltpu.async_copy(idx_hbm.at[g, pl.ds(wid * OBW, OBW)],
                             idx_v.at[par], semi.at[par])

        def stage_b(g):
            par = g % 2
            pltpu.make_async_copy(idx_hbm.at[g, pl.ds(wid * OBW, OBW)],
                                  idx_v.at[par], semi.at[par]).wait()

            def fix(t, cnt):
                v = idx_v[par, pl.ds(t * _L, _L)]
                valid = v < V
                safe_v[par, t // 8, pl.ds((t % 8) * _L, _L)] = (
                    jnp.where(valid, v, 0))
                bad = ~valid
                pos = t * _L + lanes
                bc = jnp.cumsum(jnp.where(bad, 1, 0))
                plsc.store_scatter(oov_v.at[par], [cnt + bc - 1], pos,
                                   mask=bad)
                return cnt + bc[_L - 1]

            cnt_sm[par] = lax.fori_loop(0, OBW // _L, fix, 0)
            for j in range(JT):
                pltpu.async_copy(
                    table_hbm.at[safe_v.at[par, j]],
                    rows_v.at[par, pl.ds(j * 128, 128)],
                    semg.at[par])

        def stage_c(g, first):
            par = g % 2
            cnt = cnt_sm[par]
            for j in range(JT):
                pltpu.make_async_copy(
                    table_hbm.at[safe_v.at[par, j]],
                    rows_v.at[par, pl.ds(j * 128, 128)],
                    semg.at[par]).wait()

            def zero_rows(t, _z):
                lm = (t * _L + lanes) < cnt
                p = oov_v[par, pl.ds(t * _L, _L)]
                p = jnp.where(lm, p, 0)

                def zero_col(c, _c):
                    cc = jnp.full((_L,), c, jnp.int32)
                    plsc.store_scatter(rows_v.at[par], [p, cc], zrow,
                                       mask=lm)
                    return 0

                return lax.fori_loop(0, D, zero_col, 0)

            lax.fori_loop(0, (cnt + _L - 1) // _L, zero_rows, 0)

            @pl.when(~first)
            def _drain():
                for ti in range(TI):
                    pltpu.make_async_copy(
                        trans_v.at[par, ti],
                        out_hbm.at[g, ti, pl.ds(wid * JT, JT)],
                        semo.at[par]).wait()

            @plsc.parallel_loop(0, OBW // _L, unroll=2)
            def _tpose(t):
                bvec = t * _L + lanes
                j = t // 8
                kk = (t % 8) * _L
                for d in range(D):
                    dd = jnp.full((_L,), d, jnp.int32)
                    vals = plsc.load_gather(rows_v.at[par], [bvec, dd])
                    trans_v[par, d // 8, j, d % 8, pl.ds(kk, _L)] = vals

            for ti in range(TI):
                pltpu.async_copy(trans_v.at[par, ti],
                                 out_hbm.at[g, ti, pl.ds(wid * JT, JT)],
                                 semo.at[par])

        stage_a(0)
        stage_a(1)
        stage_b(0)

        def body(g, _):
            @pl.when(g + 2 < S)
            def _a():
                stage_a(g + 2)

            @pl.when(g + 1 < S)
            def _b():
                stage_b(g + 1)

            stage_c(g, g < 2)
            return 0

        lax.fori_loop(0, S, body, 0)

        for g in (S - 2, S - 1):
            par = g % 2
            for ti in range(TI):
                pltpu.make_async_copy(
                    trans_v.at[par, ti],
                    out_hbm.at[g, ti, pl.ds(wid * JT, JT)],
                    semo.at[par]).wait()

    return lookup


def kernel(indices, table):
    V, D = table.shape
    NB, S = indices.shape
    info = plsc.get_sparse_core_info()
    idx_sm = indices.T.astype(jnp.int32)
    lookup = _make_lookup(NB, S, V, D, info.num_cores, info.num_subcores)
    tiles = lookup(idx_sm, table)
    out = tiles.transpose((2, 4, 0, 1, 3)).reshape(NB, S, D)
    return out
